# baseline (speedup 1.0000x reference)
import functools

import jax
import jax.numpy as jnp
from jax import lax
from jax.experimental import pallas as pl
from jax.experimental.pallas import tpu as pltpu
from jax.experimental.pallas import tpu_sc as plsc

N = 1024
L = 16
IB = 128
IBP = IB
JB = 512
JC = 256
NC = JC // L
TOTAL = float(N * N)

_mesh = plsc.VectorSubcoreMesh(
    core_axis_name="c", subcore_axis_name="s", num_cores=1
)


@functools.partial(
    pl.kernel,
    out_type=jax.ShapeDtypeStruct((L,), jnp.float32),
    mesh=_mesh,
    compiler_params=pltpu.CompilerParams(needs_layout_passes=False),
    scratch_types=[
        pltpu.VMEM((JB, IBP), jnp.float32),
        pltpu.VMEM((IB, JC), jnp.float32),
        pltpu.VMEM((JB,), jnp.float32),
        pltpu.VMEM((IB,), jnp.float32),
        pltpu.VMEM((IB,), jnp.float32),
        pltpu.VMEM((12, N), jnp.float32),
        pltpu.VMEM((L,), jnp.float32),
        pltpu.SemaphoreType.DMA,
        pltpu.VMEM_SHARED((8, N), jnp.float32),
        pltpu.VMEM_SHARED((2, N), jnp.float32),
        pltpu.VMEM_SHARED((2, N), jnp.float32),
    ],
)
def _edge_loss_kernel(
    rec_hbm, tgt_hbm, out_hbm,
    rec_v, tgt_v, colcnt_v, s_v, rowcnt_v, fin_v, out_v, dma_sem,
    sh_colcnt, sh_s, sh_rowcnt,
):
    s = lax.axis_index("s")
    p = s // 2
    q = s % 2
    i0 = p * IB
    j0 = q * JB
    zeros = jnp.zeros((L,), jnp.float32)
    iota = lax.iota(jnp.int32, L)

    rec_dma = pltpu.async_copy(
        rec_hbm.at[pl.ds(j0, JB), pl.ds(i0, IB)],
        rec_v.at[:, pl.ds(0, IB)],
        dma_sem,
    )
    pltpu.sync_copy(tgt_hbm.at[pl.ds(i0, IB), pl.ds(j0, JC)], tgt_v)
    rec_dma.wait()

    for h in range(JB // JC):
        if h > 0:
            pltpu.sync_copy(
                tgt_hbm.at[pl.ds(i0, IB), pl.ds(j0 + h * JC, JC)], tgt_v
            )

        def row_body(a, carry):
            sgrp, rgrp = carry[0], carry[1]
            ccnt = carry[2]
            col_idx = jnp.full((L,), a, jnp.int32)
            acc = zeros
            rowacc = zeros
            ncnt = []
            for jc in range(NC):
                jl = jc * L
                t = tgt_v[a, pl.ds(jl, L)]
                r = plsc.load_gather(rec_v, [h * JC + jl + iota, col_idx])
                d = r - t
                nz = jnp.where(t != 0.0, 1.0, 0.0).astype(jnp.float32)
                acc = acc + d * d
                rowacc = rowacc + nz
                ncnt.append(ccnt[jc] + nz)
            gslot = (a // L) * L
            sel = iota == (a % L)
            sgrp = jnp.where(sel, jnp.sum(acc), sgrp)
            rgrp = jnp.where(sel, jnp.sum(rowacc), rgrp)
            s_v[pl.ds(gslot, L)] = sgrp
            rowcnt_v[pl.ds(gslot, L)] = rgrp
            return (sgrp, rgrp, tuple(ncnt))

        init = (zeros, zeros, tuple(zeros for _ in range(NC)))
        _, _, ccnt = lax.fori_loop(0, IB, row_body, init)
        for jc in range(NC):
            colcnt_v[pl.ds(h * JC + jc * L, L)] = ccnt[jc]

    pltpu.sync_copy(s_v, sh_s.at[q, pl.ds(i0, IB)])
    pltpu.sync_copy(rowcnt_v, sh_rowcnt.at[q, pl.ds(i0, IB)])
    pltpu.sync_copy(colcnt_v, sh_colcnt.at[p, pl.ds(j0, JB)])
    plsc.subcore_barrier()

    @pl.when(s == 0)
    def _final():
        pltpu.sync_copy(sh_colcnt, fin_v.at[pl.ds(0, 8)])
        pltpu.sync_copy(sh_s, fin_v.at[pl.ds(8, 2)])
        pltpu.sync_copy(sh_rowcnt, fin_v.at[pl.ds(10, 2)])

        def comb_body(c, carry):
            e_acc, t_acc, ts_acc = carry
            cl = c * L
            col = fin_v[0, pl.ds(cl, L)]
            for r in range(1, 8):
                col = col + fin_v[r, pl.ds(cl, L)]
            s16 = fin_v[8, pl.ds(cl, L)] + fin_v[9, pl.ds(cl, L)]
            row16 = fin_v[10, pl.ds(cl, L)] + fin_v[11, pl.ds(cl, L)]
            is_set = (row16 > 0.0) | (col > 0.0)
            ts_acc = ts_acc + jnp.where(is_set, s16, 0.0)
            return e_acc + col, t_acc + s16, ts_acc

        e_v, t_v, ts_v = lax.fori_loop(
            0, N // L, comb_body, (zeros, zeros, zeros)
        )
        ones = jnp.full((L,), 1.0, jnp.float32)
        e16 = ones * jnp.sum(e_v)
        t16 = ones * jnp.sum(t_v)
        ts16 = ones * jnp.sum(ts_v)
        neg_w = jnp.where(e16 >= TOTAL, 0.0, e16 / (TOTAL - e16))
        out_v[...] = ts16 + neg_w * (t16 - ts16)
        pltpu.sync_copy(out_v, out_hbm)


def kernel(adj_rec, adj_tgt):
    out = _edge_loss_kernel(adj_rec, adj_tgt)
    return out[0]


# speedup vs baseline: 1.5620x; 1.5620x over previous
import functools

import jax
import jax.numpy as jnp
from jax import lax
from jax.experimental import pallas as pl
from jax.experimental.pallas import tpu as pltpu
from jax.experimental.pallas import tpu_sc as plsc

N = 1024
L = 16
IB = 128
JB = 512
JC = 256
NC = JC // L
TOTAL = float(N * N)

_GATHER_DNUMS = lax.GatherDimensionNumbers(
    offset_dims=(), collapsed_slice_dims=(0,), start_index_map=(0,)
)

_mesh = plsc.VectorSubcoreMesh(
    core_axis_name="c", subcore_axis_name="s", num_cores=1
)


@functools.partial(
    pl.kernel,
    out_type=jax.ShapeDtypeStruct((L,), jnp.float32),
    mesh=_mesh,
    compiler_params=pltpu.CompilerParams(needs_layout_passes=False),
    scratch_types=[
        pltpu.VMEM((JB, IB), jnp.float32),
        pltpu.VMEM((IB, JC), jnp.float32),
        pltpu.VMEM((JB,), jnp.float32),
        pltpu.VMEM((IB,), jnp.float32),
        pltpu.VMEM((IB,), jnp.float32),
        pltpu.VMEM((12, N), jnp.float32),
        pltpu.VMEM((L,), jnp.float32),
        pltpu.SemaphoreType.DMA,
        pltpu.VMEM_SHARED((8, N), jnp.float32),
        pltpu.VMEM_SHARED((2, N), jnp.float32),
        pltpu.VMEM_SHARED((2, N), jnp.float32),
    ],
)
def _edge_loss_kernel(
    rec_hbm, tgt_hbm, out_hbm,
    rec_v, tgt_v, colcnt_v, s_v, rowcnt_v, fin_v, out_v, dma_sem,
    sh_colcnt, sh_s, sh_rowcnt,
):
    s = lax.axis_index("s")
    p = s // 2
    q = s % 2
    i0 = p * IB
    j0 = q * JB
    zeros = jnp.zeros((L,), jnp.float32)
    iota = lax.iota(jnp.int32, L)

    rec_dma = pltpu.async_copy(
        rec_hbm.at[pl.ds(j0, JB), pl.ds(i0, IB)], rec_v, dma_sem
    )
    pltpu.sync_copy(tgt_hbm.at[pl.ds(i0, IB), pl.ds(j0, JC)], tgt_v)
    rec_dma.wait()

    for h in range(JB // JC):
        hbase = h * JC
        if h > 0:
            pltpu.sync_copy(
                tgt_hbm.at[pl.ds(i0, IB), pl.ds(j0 + hbase, JC)], tgt_v
            )

        def ab_body(ab, ccnt):
            i_idx = ab * L + iota

            def d_body(d, carry):
                acc, rowacc = carry[0], carry[1]
                cc = carry[2]
                perm = (iota + d) & (L - 1)
                inv = (iota - d) & (L - 1)
                ncc = []
                for jc in range(NC):
                    tj = jc * L + perm
                    t = plsc.load_gather(tgt_v, [i_idx, tj])
                    rj = tj + hbase if h else tj
                    r = plsc.load_gather(rec_v, [rj, i_idx])
                    dd = r - t
                    acc = acc + dd * dd
                    nz = jnp.where(t != 0.0, 1.0, 0.0).astype(jnp.float32)
                    rowacc = rowacc + nz
                    nzu = lax.gather(
                        nz,
                        inv[:, None],
                        _GATHER_DNUMS,
                        (1,),
                        mode=lax.GatherScatterMode.PROMISE_IN_BOUNDS,
                    )
                    ncc.append(cc[jc] + nzu)
                return (acc, rowacc, tuple(ncc))

            acc, rowacc, ccnt = lax.fori_loop(
                0, L, d_body, (zeros, zeros, ccnt)
            )
            sl = pl.ds(ab * L, L)
            if h == 0:
                s_v[sl] = acc
                rowcnt_v[sl] = rowacc
            else:
                s_v[sl] = s_v[sl] + acc
                rowcnt_v[sl] = rowcnt_v[sl] + rowacc
            return ccnt

        ccnt = lax.fori_loop(
            0, IB // L, ab_body, tuple(zeros for _ in range(NC))
        )
        for jc in range(NC):
            colcnt_v[pl.ds(hbase + jc * L, L)] = ccnt[jc]

    pltpu.sync_copy(s_v, sh_s.at[q, pl.ds(i0, IB)])
    pltpu.sync_copy(rowcnt_v, sh_rowcnt.at[q, pl.ds(i0, IB)])
    pltpu.sync_copy(colcnt_v, sh_colcnt.at[p, pl.ds(j0, JB)])
    plsc.subcore_barrier()

    @pl.when(s == 0)
    def _final():
        pltpu.sync_copy(sh_colcnt, fin_v.at[pl.ds(0, 8)])
        pltpu.sync_copy(sh_s, fin_v.at[pl.ds(8, 2)])
        pltpu.sync_copy(sh_rowcnt, fin_v.at[pl.ds(10, 2)])

        def comb_body(c, carry):
            e_acc, t_acc, ts_acc = carry
            cl = c * L
            col = fin_v[0, pl.ds(cl, L)]
            for r in range(1, 8):
                col = col + fin_v[r, pl.ds(cl, L)]
            s16 = fin_v[8, pl.ds(cl, L)] + fin_v[9, pl.ds(cl, L)]
            row16 = fin_v[10, pl.ds(cl, L)] + fin_v[11, pl.ds(cl, L)]
            is_set = (row16 > 0.0) | (col > 0.0)
            ts_acc = ts_acc + jnp.where(is_set, s16, 0.0)
            return e_acc + col, t_acc + s16, ts_acc

        e_v, t_v, ts_v = lax.fori_loop(
            0, N // L, comb_body, (zeros, zeros, zeros)
        )
        ones = jnp.full((L,), 1.0, jnp.float32)
        e16 = ones * jnp.sum(e_v)
        t16 = ones * jnp.sum(t_v)
        ts16 = ones * jnp.sum(ts_v)
        neg_w = jnp.where(e16 >= TOTAL, 0.0, e16 / (TOTAL - e16))
        out_v[...] = ts16 + neg_w * (t16 - ts16)
        pltpu.sync_copy(out_v, out_hbm)


def kernel(adj_rec, adj_tgt):
    out = _edge_loss_kernel(adj_rec, adj_tgt)
    return out[0]


# speedup vs baseline: 1.8366x; 1.1758x over previous
import functools

import jax
import jax.numpy as jnp
from jax import lax
from jax.experimental import pallas as pl
from jax.experimental.pallas import tpu as pltpu
from jax.experimental.pallas import tpu_sc as plsc

N = 1024
L = 16
IB = 128
JB = 512
JC = 256
NC = JC // L
TOTAL = float(N * N)

_GATHER_DNUMS = lax.GatherDimensionNumbers(
    offset_dims=(), collapsed_slice_dims=(0,), start_index_map=(0,)
)

_mesh = plsc.VectorSubcoreMesh(
    core_axis_name="c", subcore_axis_name="s", num_cores=1
)


@functools.partial(
    pl.kernel,
    out_type=jax.ShapeDtypeStruct((L,), jnp.float32),
    mesh=_mesh,
    compiler_params=pltpu.CompilerParams(needs_layout_passes=False),
    scratch_types=[
        pltpu.VMEM((JB, IB), jnp.float32),
        pltpu.VMEM((IB, JC), jnp.float32),
        pltpu.VMEM((JB,), jnp.float32),
        pltpu.VMEM((IB,), jnp.float32),
        pltpu.VMEM((IB,), jnp.float32),
        pltpu.VMEM((12, N), jnp.float32),
        pltpu.VMEM((L,), jnp.float32),
        pltpu.SemaphoreType.DMA,
        pltpu.VMEM_SHARED((8, N), jnp.float32),
        pltpu.VMEM_SHARED((2, N), jnp.float32),
        pltpu.VMEM_SHARED((2, N), jnp.float32),
    ],
)
def _edge_loss_kernel(
    rec_hbm, tgt_hbm, out_hbm,
    rec_v, tgt_v, colcnt_v, s_v, rowcnt_v, fin_v, out_v, dma_sem,
    sh_colcnt, sh_s, sh_rowcnt,
):
    s = lax.axis_index("s")
    p = s // 2
    q = s % 2
    i0 = p * IB
    j0 = q * JB
    zeros = jnp.zeros((L,), jnp.float32)
    iota = lax.iota(jnp.int32, L)

    rec_dma = pltpu.async_copy(
        rec_hbm.at[pl.ds(j0, JB), pl.ds(i0, IB)], rec_v, dma_sem
    )
    pltpu.sync_copy(tgt_hbm.at[pl.ds(i0, IB), pl.ds(j0, JC)], tgt_v)
    rec_dma.wait()

    for h in range(JB // JC):
        hbase = h * JC
        if h > 0:
            pltpu.sync_copy(
                tgt_hbm.at[pl.ds(i0, IB), pl.ds(j0 + hbase, JC)], tgt_v
            )

        def ab_body(ab, _):
            i_idx = ab * L + iota

            def jc_body(jc, carry):
                acc, rowacc = carry
                jl = jc * L
                tgt_c = tgt_v.at[pl.ds(ab * L, L), :]
                rec_c = rec_v.at[pl.ds(hbase + jl, L), :]
                ccacc = zeros
                for d in range(L):
                    perm = (iota + d) & (L - 1)
                    inv = (iota - d) & (L - 1)
                    t = plsc.load_gather(tgt_c, [iota, jl + perm])
                    r = plsc.load_gather(rec_c, [perm, i_idx])
                    dd = r - t
                    acc = acc + dd * dd
                    nz = jnp.where(t != 0.0, 1.0, 0.0).astype(jnp.float32)
                    rowacc = rowacc + nz
                    nzu = lax.gather(
                        nz,
                        inv[:, None],
                        _GATHER_DNUMS,
                        (1,),
                        mode=lax.GatherScatterMode.PROMISE_IN_BOUNDS,
                    )
                    ccacc = ccacc + nzu
                cl = pl.ds(hbase + jl, L)
                colcnt_v[cl] = jnp.where(ab == 0, 0.0, colcnt_v[cl]) + ccacc
                return acc, rowacc

            acc, rowacc = lax.fori_loop(0, NC, jc_body, (zeros, zeros))
            sl = pl.ds(ab * L, L)
            if h == 0:
                s_v[sl] = acc
                rowcnt_v[sl] = rowacc
            else:
                s_v[sl] = s_v[sl] + acc
                rowcnt_v[sl] = rowcnt_v[sl] + rowacc
            return 0

        lax.fori_loop(0, IB // L, ab_body, 0)

    pltpu.sync_copy(s_v, sh_s.at[q, pl.ds(i0, IB)])
    pltpu.sync_copy(rowcnt_v, sh_rowcnt.at[q, pl.ds(i0, IB)])
    pltpu.sync_copy(colcnt_v, sh_colcnt.at[p, pl.ds(j0, JB)])
    plsc.subcore_barrier()

    @pl.when(s == 0)
    def _final():
        pltpu.sync_copy(sh_colcnt, fin_v.at[pl.ds(0, 8)])
        pltpu.sync_copy(sh_s, fin_v.at[pl.ds(8, 2)])
        pltpu.sync_copy(sh_rowcnt, fin_v.at[pl.ds(10, 2)])

        def comb_body(c, carry):
            e_acc, t_acc, ts_acc = carry
            cl = c * L
            col = fin_v[0, pl.ds(cl, L)]
            for r in range(1, 8):
                col = col + fin_v[r, pl.ds(cl, L)]
            s16 = fin_v[8, pl.ds(cl, L)] + fin_v[9, pl.ds(cl, L)]
            row16 = fin_v[10, pl.ds(cl, L)] + fin_v[11, pl.ds(cl, L)]
            is_set = (row16 > 0.0) | (col > 0.0)
            ts_acc = ts_acc + jnp.where(is_set, s16, 0.0)
            return e_acc + col, t_acc + s16, ts_acc

        e_v, t_v, ts_v = lax.fori_loop(
            0, N // L, comb_body, (zeros, zeros, zeros)
        )
        ones = jnp.full((L,), 1.0, jnp.float32)
        e16 = ones * jnp.sum(e_v)
        t16 = ones * jnp.sum(t_v)
        ts16 = ones * jnp.sum(ts_v)
        neg_w = jnp.where(e16 >= TOTAL, 0.0, e16 / (TOTAL - e16))
        out_v[...] = ts16 + neg_w * (t16 - ts16)
        pltpu.sync_copy(out_v, out_hbm)


def kernel(adj_rec, adj_tgt):
    out = _edge_loss_kernel(adj_rec, adj_tgt)
    return out[0]


# speedup vs baseline: 1.9092x; 1.0395x over previous
import functools

import jax
import jax.numpy as jnp
from jax import lax
from jax.experimental import pallas as pl
from jax.experimental.pallas import tpu as pltpu
from jax.experimental.pallas import tpu_sc as plsc

N = 1024
L = 16
IB = 128
JB = 512
JC = 128
NC = JC // L
NPH = JB // JC
RCH = JB // NPH
TOTAL = float(N * N)

_GATHER_DNUMS = lax.GatherDimensionNumbers(
    offset_dims=(), collapsed_slice_dims=(0,), start_index_map=(0,)
)

_mesh = plsc.VectorSubcoreMesh(
    core_axis_name="c", subcore_axis_name="s", num_cores=1
)


@functools.partial(
    pl.kernel,
    out_type=jax.ShapeDtypeStruct((L,), jnp.float32),
    mesh=_mesh,
    compiler_params=pltpu.CompilerParams(needs_layout_passes=False),
    scratch_types=[
        pltpu.VMEM((JB, IB), jnp.float32),
        pltpu.VMEM((2, IB, JC), jnp.float32),
        pltpu.VMEM((JB,), jnp.float32),
        pltpu.VMEM((IB,), jnp.float32),
        pltpu.VMEM((IB,), jnp.float32),
        pltpu.VMEM((12, N), jnp.float32),
        pltpu.VMEM((L,), jnp.float32),
        pltpu.SemaphoreType.DMA,
        pltpu.SemaphoreType.DMA,
        pltpu.SemaphoreType.DMA,
        pltpu.VMEM_SHARED((8, N), jnp.float32),
        pltpu.VMEM_SHARED((2, N), jnp.float32),
        pltpu.VMEM_SHARED((2, N), jnp.float32),
    ],
)
def _edge_loss_kernel(
    rec_hbm, tgt_hbm, out_hbm,
    rec_v, tgt_v, colcnt_v, s_v, rowcnt_v, fin_v, out_v,
    rec_sem, tgt_sem0, tgt_sem1, sh_colcnt, sh_s, sh_rowcnt,
):
    s = lax.axis_index("s")
    p = s // 2
    q = s % 2
    i0 = p * IB
    j0 = q * JB
    zeros = jnp.zeros((L,), jnp.float32)
    iota = lax.iota(jnp.int32, L)

    tgt_sems = [tgt_sem0, tgt_sem1]

    def tgt_copy(h):
        return pltpu.async_copy(
            tgt_hbm.at[pl.ds(i0, IB), pl.ds(j0 + h * JC, JC)],
            tgt_v.at[h % 2], tgt_sems[h % 2],
        )

    tgt_dma = tgt_copy(0)
    rec_dmas = [
        pltpu.async_copy(
            rec_hbm.at[pl.ds(j0 + h * RCH, RCH), pl.ds(i0, IB)],
            rec_v.at[pl.ds(h * RCH, RCH), :], rec_sem,
        )
        for h in range(NPH)
    ]

    for h in range(NPH):
        hbase = h * JC
        b = h % 2
        tgt_dma.wait()
        rec_dmas[h].wait()
        if h + 1 < NPH:
            tgt_dma = tgt_copy(h + 1)

        def ab_body(ab, _):
            i_idx = ab * L + iota

            def jc_body(jc, carry):
                acc, rowacc = carry
                jl = jc * L
                tgt_c = tgt_v.at[b, pl.ds(ab * L, L), :]
                rec_c = rec_v.at[pl.ds(hbase + jl, L), :]
                ccacc = zeros
                for d in range(L):
                    perm = (iota + d) & (L - 1)
                    inv = (iota - d) & (L - 1)
                    t = plsc.load_gather(tgt_c, [iota, jl + perm])
                    r = plsc.load_gather(rec_c, [perm, i_idx])
                    dd = r - t
                    acc = acc + dd * dd
                    nz = jnp.where(t != 0.0, 1.0, 0.0).astype(jnp.float32)
                    rowacc = rowacc + nz
                    nzu = lax.gather(
                        nz,
                        inv[:, None],
                        _GATHER_DNUMS,
                        (1,),
                        mode=lax.GatherScatterMode.PROMISE_IN_BOUNDS,
                    )
                    ccacc = ccacc + nzu
                cl = pl.ds(hbase + jl, L)
                colcnt_v[cl] = jnp.where(ab == 0, 0.0, colcnt_v[cl]) + ccacc
                return acc, rowacc

            acc, rowacc = lax.fori_loop(0, NC, jc_body, (zeros, zeros))
            sl = pl.ds(ab * L, L)
            if h == 0:
                s_v[sl] = acc
                rowcnt_v[sl] = rowacc
            else:
                s_v[sl] = s_v[sl] + acc
                rowcnt_v[sl] = rowcnt_v[sl] + rowacc
            return 0

        lax.fori_loop(0, IB // L, ab_body, 0)

    pltpu.sync_copy(s_v, sh_s.at[q, pl.ds(i0, IB)])
    pltpu.sync_copy(rowcnt_v, sh_rowcnt.at[q, pl.ds(i0, IB)])
    pltpu.sync_copy(colcnt_v, sh_colcnt.at[p, pl.ds(j0, JB)])
    plsc.subcore_barrier()

    @pl.when(s == 0)
    def _final():
        pltpu.sync_copy(sh_colcnt, fin_v.at[pl.ds(0, 8)])
        pltpu.sync_copy(sh_s, fin_v.at[pl.ds(8, 2)])
        pltpu.sync_copy(sh_rowcnt, fin_v.at[pl.ds(10, 2)])

        def comb_body(c, carry):
            e_acc, t_acc, ts_acc = carry
            cl = c * L
            col = fin_v[0, pl.ds(cl, L)]
            for r in range(1, 8):
                col = col + fin_v[r, pl.ds(cl, L)]
            s16 = fin_v[8, pl.ds(cl, L)] + fin_v[9, pl.ds(cl, L)]
            row16 = fin_v[10, pl.ds(cl, L)] + fin_v[11, pl.ds(cl, L)]
            is_set = (row16 > 0.0) | (col > 0.0)
            ts_acc = ts_acc + jnp.where(is_set, s16, 0.0)
            return e_acc + col, t_acc + s16, ts_acc

        e_v, t_v, ts_v = lax.fori_loop(
            0, N // L, comb_body, (zeros, zeros, zeros)
        )
        ones = jnp.full((L,), 1.0, jnp.float32)
        e16 = ones * jnp.sum(e_v)
        t16 = ones * jnp.sum(t_v)
        ts16 = ones * jnp.sum(ts_v)
        neg_w = jnp.where(e16 >= TOTAL, 0.0, e16 / (TOTAL - e16))
        out_v[...] = ts16 + neg_w * (t16 - ts16)
        pltpu.sync_copy(out_v, out_hbm)


def kernel(adj_rec, adj_tgt):
    out = _edge_loss_kernel(adj_rec, adj_tgt)
    return out[0]
